# baseline (speedup 1.0000x reference)
import functools

import jax
import jax.numpy as jnp
from jax import lax
from jax.experimental import pallas as pl
from jax.experimental.pallas import tpu as pltpu
from jax.experimental.pallas import tpu_sc as plsc

_N = 10000
_D = 256
_B = 1024
_K = 32
_R = 3
_P = 16
_E = 1024
_C = 2

_NC, _NS, _L = 2, 16, 16
_NW = _NC * _NS

_NEIGH = _R * _B * _K
_NB_W = _NEIGH // _NW
_ND_W = _B // _NW
_SEG = _R * _B
_SEG_W = _SEG // _NW
_SEG_CHUNK = 8
_CHUNKS = _SEG_W // _SEG_CHUNK

def _sc_mesh():
    return plsc.VectorSubcoreMesh(
        core_axis_name="c", subcore_axis_name="s",
        num_cores=_NC, num_subcores=_NS)


_SC_PARAMS = pltpu.CompilerParams(
    needs_layout_passes=False, use_tc_tiling_on_sc=False)


def _scores_body(feat_ref, wl_ref, out_ref):
    out_ref[...] = lax.dot_general(wl_ref[...], feat_ref[...],
                                   (((0,), (1,)), ((), ())),
                                   preferred_element_type=jnp.float32)


def _all_scores(features, W_label):
    return pl.pallas_call(
        _scores_body,
        out_shape=jax.ShapeDtypeStruct((_C, _N), jnp.float32),
    )(features, W_label)


def _g1_body(scores_hbm, neigh_hbm, nodes_hbm, feat_hbm,
             nsc_out, ssc_out, srow_out,
             tbl0, tbl1, nidx, nval, sidx, sval, srows, sem):
    wid = lax.axis_index("s") * _NC + lax.axis_index("c")
    pltpu.sync_copy(scores_hbm.at[0], tbl0)
    pltpu.sync_copy(scores_hbm.at[1], tbl1)
    pltpu.sync_copy(neigh_hbm.at[pl.ds(wid * _NB_W, _NB_W)], nidx)
    pltpu.sync_copy(nodes_hbm.at[pl.ds(wid * _ND_W, _ND_W)], sidx)
    cp = pltpu.async_copy(feat_hbm.at[sidx], srows, sem)

    def nstep(i, carry):
        ids = nidx[pl.ds(i * _L, _L)]
        nval[pl.ds(i * _L, _L)] = plsc.load_gather(tbl1, [ids])
        return carry
    lax.fori_loop(0, _NB_W // _L, nstep, 0)

    def sstep(i, carry):
        ids = sidx[pl.ds(i * _L, _L)]
        sval[0, pl.ds(i * _L, _L)] = plsc.load_gather(tbl0, [ids])
        sval[1, pl.ds(i * _L, _L)] = plsc.load_gather(tbl1, [ids])
        return carry
    lax.fori_loop(0, _ND_W // _L, sstep, 0)

    cp.wait()
    pltpu.sync_copy(nval, nsc_out.at[pl.ds(wid * _NB_W, _NB_W)])
    pltpu.sync_copy(sval, ssc_out.at[wid])
    pltpu.sync_copy(srows, srow_out.at[pl.ds(wid * _ND_W, _ND_W)])


def _g1_call(all_scores, neigh_flat, nodes, features):
    fn = pl.kernel(
        _g1_body,
        out_type=(
            jax.ShapeDtypeStruct((_NEIGH,), jnp.float32),
            jax.ShapeDtypeStruct((_NW, 2, _ND_W), jnp.float32),
            jax.ShapeDtypeStruct((_B, _D), jnp.float32),
        ),
        mesh=_sc_mesh(),
        scratch_types=[
            pltpu.VMEM((_N,), jnp.float32),
            pltpu.VMEM((_N,), jnp.float32),
            pltpu.VMEM((_NB_W,), jnp.int32),
            pltpu.VMEM((_NB_W,), jnp.float32),
            pltpu.VMEM((_ND_W,), jnp.int32),
            pltpu.VMEM((2, _ND_W), jnp.float32),
            pltpu.VMEM((_ND_W, _D), jnp.float32),
            pltpu.SemaphoreType.DMA,
        ],
        compiler_params=_SC_PARAMS,
    )
    return fn(all_scores, neigh_flat, nodes, features)


def _select_body(ns_ref, ss_ref, ni_ref, out_ref):
    d = jnp.abs(ns_ref[...] - ss_ref[...][None])
    kio = lax.broadcasted_iota(jnp.int32, (_R, _K, _B), 1)
    rank = jnp.zeros((_R, _K, _B), jnp.int32)
    for j in range(_K):
        dj = lax.slice_in_dim(d, j, j + 1, axis=1)
        before = (dj < d) | ((dj == d) & (j < kio))
        rank = rank + before.astype(jnp.int32)
    mask = rank < _P
    pos = jnp.zeros((_R, _K, _B), jnp.int32)
    for j in range(_K):
        mj = lax.slice_in_dim(mask, j, j + 1, axis=1)
        pos = pos + (mj & (j < kio)).astype(jnp.int32)
    pio = lax.broadcasted_iota(jnp.int32, (_R, _P, _B), 1)
    idsf = ni_ref[...].astype(jnp.float32)
    ch = jnp.zeros((_R, _P, _B), jnp.float32)
    for k in range(_K):
        sel = ((lax.slice_in_dim(pos, k, k + 1, axis=1) == pio) &
               lax.slice_in_dim(mask, k, k + 1, axis=1))
        ch = ch + lax.slice_in_dim(idsf, k, k + 1, axis=1) * sel.astype(jnp.float32)
    out_ref[...] = jnp.transpose(ch, (0, 2, 1)).astype(jnp.int32)


def _select_call(ns3, sself, neigh_t):
    return pl.pallas_call(
        _select_body,
        out_shape=jax.ShapeDtypeStruct((_R, _B, _P), jnp.int32),
    )(ns3, sself, neigh_t)


_NBUF = 3


def _g2_body(chosen_hbm, feat_hbm, agg_out,
             cidx_a, cidx_b, cidx_c, rows_a, rows_b, rows_c, aggc,
             sem_a, sem_b, sem_c):
    wid = lax.axis_index("s") * _NC + lax.axis_index("c")
    nrow = _SEG_CHUNK * _P
    bufs = ((cidx_a, rows_a, sem_a), (cidx_b, rows_b, sem_b),
            (cidx_c, rows_c, sem_c))

    def start(c):
        cidx, rows, sem = bufs[c % _NBUF]
        base = wid * _SEG_W * _P + c * nrow
        pltpu.sync_copy(chosen_hbm.at[pl.ds(base, nrow)], cidx)
        return pltpu.async_copy(feat_hbm.at[cidx], rows, sem)

    cps = [start(0), start(1)]
    for c in range(_CHUNKS):
        cps[0].wait()
        cps.pop(0)
        if c + _NBUF - 1 < _CHUNKS:
            cps.append(start(c + _NBUF - 1))
        rows = bufs[c % _NBUF][1]

        def seg(s, carry2):
            def jstep(j, carry3):
                acc = rows[s * _P, pl.ds(j * _L, _L)]
                for p in range(1, _P):
                    acc = acc + rows[s * _P + p, pl.ds(j * _L, _L)]
                aggc[s, pl.ds(j * _L, _L)] = acc * (1.0 / _P)
                return carry3
            return lax.fori_loop(0, _D // _L, jstep, carry2)
        lax.fori_loop(0, _SEG_CHUNK, seg, 0)
        pltpu.sync_copy(aggc, agg_out.at[pl.ds(wid * _SEG_W + c * _SEG_CHUNK,
                                               _SEG_CHUNK)])


def _g2_call(chosen_flat, features):
    fn = pl.kernel(
        _g2_body,
        out_type=jax.ShapeDtypeStruct((_SEG, _D), jnp.float32),
        mesh=_sc_mesh(),
        scratch_types=[
            pltpu.VMEM((_SEG_CHUNK * _P,), jnp.int32),
            pltpu.VMEM((_SEG_CHUNK * _P,), jnp.int32),
            pltpu.VMEM((_SEG_CHUNK * _P,), jnp.int32),
            pltpu.VMEM((_SEG_CHUNK * _P, _D), jnp.float32),
            pltpu.VMEM((_SEG_CHUNK * _P, _D), jnp.float32),
            pltpu.VMEM((_SEG_CHUNK * _P, _D), jnp.float32),
            pltpu.VMEM((_SEG_CHUNK, _D), jnp.float32),
            pltpu.SemaphoreType.DMA,
            pltpu.SemaphoreType.DMA,
            pltpu.SemaphoreType.DMA,
        ],
        compiler_params=_SC_PARAMS,
    )
    return fn(chosen_flat, features)


def _final_body(self_ref, agg_ref, wi_ref, wn_ref, wt_ref, out_ref):
    sf = self_ref[...]
    h = jnp.dot(sf, wn_ref[0:_D, :], preferred_element_type=jnp.float32)
    for r in range(_R):
        wr = wi_ref[r]
        ir = jnp.dot(sf, wr[0:_D, :], preferred_element_type=jnp.float32)
        ir = ir + jnp.dot(agg_ref[r], wr[_D:2 * _D, :],
                          preferred_element_type=jnp.float32)
        ir = jnp.maximum(ir, 0.0)
        h = h + jnp.dot(ir, wn_ref[_D + r * _E:_D + (r + 1) * _E, :],
                        preferred_element_type=jnp.float32)
    inter = jnp.maximum(h, 0.0)
    out_ref[...] = lax.dot_general(inter, wt_ref[...],
                                   (((1,), (1,)), ((), ())),
                                   preferred_element_type=jnp.float32)


def _final_call(self_rows, agg3, W_intra, W_inter, weight):
    return pl.pallas_call(
        _final_body,
        out_shape=jax.ShapeDtypeStruct((_B, _C), jnp.float32),
    )(self_rows, agg3, W_intra, W_inter, weight)


def kernel(nodes, labels, neigh_idx, features, train_pos,
           W_label, W_intra, W_inter, weight):
    nodes = nodes.astype(jnp.int32)
    neigh_idx = neigh_idx.astype(jnp.int32)
    all_scores = _all_scores(features, W_label)
    neigh_t = neigh_idx.transpose(0, 2, 1)
    neigh_flat = neigh_t.reshape(-1)
    nsc, ssc, self_rows = _g1_call(all_scores, neigh_flat, nodes, features)
    label_scores = ssc.transpose(0, 2, 1).reshape(_B, 2)
    sself = label_scores[:, 1].reshape(1, _B)
    ns3 = nsc.reshape(_R, _K, _B)
    chosen = _select_call(ns3, sself, neigh_t)
    agg = _g2_call(chosen.reshape(-1), features)
    scores = _final_call(self_rows, agg.reshape(_R, _B, _D),
                         W_intra, W_inter, weight)
    return scores, label_scores


# speedup vs baseline: 1.0251x; 1.0251x over previous
import functools

import jax
import jax.numpy as jnp
from jax import lax
from jax.experimental import pallas as pl
from jax.experimental.pallas import tpu as pltpu
from jax.experimental.pallas import tpu_sc as plsc

_N = 10000
_D = 256
_B = 1024
_K = 32
_R = 3
_P = 16
_E = 1024
_C = 2

_NC, _NS, _L = 2, 16, 16
_NW = _NC * _NS

_NEIGH = _R * _B * _K
_NB_W = _NEIGH // _NW
_ND_W = _B // _NW
_SEG = _R * _B
_SEG_W = _SEG // _NW
_SEG_CHUNK = 12
_CHUNKS = _SEG_W // _SEG_CHUNK

def _sc_mesh():
    return plsc.VectorSubcoreMesh(
        core_axis_name="c", subcore_axis_name="s",
        num_cores=_NC, num_subcores=_NS)


_SC_PARAMS = pltpu.CompilerParams(
    needs_layout_passes=False, use_tc_tiling_on_sc=False)


def _scores_body(feat_ref, wl_ref, out_ref):
    out_ref[...] = lax.dot_general(wl_ref[...], feat_ref[...],
                                   (((0,), (1,)), ((), ())),
                                   preferred_element_type=jnp.float32)


def _all_scores(features, W_label):
    return pl.pallas_call(
        _scores_body,
        out_shape=jax.ShapeDtypeStruct((_C, _N), jnp.float32),
    )(features, W_label)


def _g1_body(scores_hbm, neigh_hbm, nodes_hbm, feat_hbm,
             nsc_out, ssc_out, srow_out,
             tbl0, tbl1, nidx, nval, sidx, sval, srows, sem):
    wid = lax.axis_index("s") * _NC + lax.axis_index("c")
    pltpu.sync_copy(scores_hbm.at[0], tbl0)
    pltpu.sync_copy(scores_hbm.at[1], tbl1)
    pltpu.sync_copy(neigh_hbm.at[pl.ds(wid * _NB_W, _NB_W)], nidx)
    pltpu.sync_copy(nodes_hbm.at[pl.ds(wid * _ND_W, _ND_W)], sidx)
    cp = pltpu.async_copy(feat_hbm.at[sidx], srows, sem)

    def nstep(i, carry):
        ids = nidx[pl.ds(i * _L, _L)]
        nval[pl.ds(i * _L, _L)] = plsc.load_gather(tbl1, [ids])
        return carry
    lax.fori_loop(0, _NB_W // _L, nstep, 0)

    def sstep(i, carry):
        ids = sidx[pl.ds(i * _L, _L)]
        sval[0, pl.ds(i * _L, _L)] = plsc.load_gather(tbl0, [ids])
        sval[1, pl.ds(i * _L, _L)] = plsc.load_gather(tbl1, [ids])
        return carry
    lax.fori_loop(0, _ND_W // _L, sstep, 0)

    cp.wait()
    pltpu.sync_copy(nval, nsc_out.at[pl.ds(wid * _NB_W, _NB_W)])
    pltpu.sync_copy(sval, ssc_out.at[wid])
    pltpu.sync_copy(srows, srow_out.at[pl.ds(wid * _ND_W, _ND_W)])


def _g1_call(all_scores, neigh_flat, nodes, features):
    fn = pl.kernel(
        _g1_body,
        out_type=(
            jax.ShapeDtypeStruct((_NEIGH,), jnp.float32),
            jax.ShapeDtypeStruct((_NW, 2, _ND_W), jnp.float32),
            jax.ShapeDtypeStruct((_B, _D), jnp.float32),
        ),
        mesh=_sc_mesh(),
        scratch_types=[
            pltpu.VMEM((_N,), jnp.float32),
            pltpu.VMEM((_N,), jnp.float32),
            pltpu.VMEM((_NB_W,), jnp.int32),
            pltpu.VMEM((_NB_W,), jnp.float32),
            pltpu.VMEM((_ND_W,), jnp.int32),
            pltpu.VMEM((2, _ND_W), jnp.float32),
            pltpu.VMEM((_ND_W, _D), jnp.float32),
            pltpu.SemaphoreType.DMA,
        ],
        compiler_params=_SC_PARAMS,
    )
    return fn(all_scores, neigh_flat, nodes, features)


def _select_body(ns_ref, ss_ref, ni_ref, out_ref):
    d = jnp.abs(ns_ref[...] - ss_ref[...][None])
    kio = lax.broadcasted_iota(jnp.int32, (_R, _K, _B), 1)
    rank = jnp.zeros((_R, _K, _B), jnp.int32)
    for j in range(_K):
        dj = lax.slice_in_dim(d, j, j + 1, axis=1)
        before = (dj < d) | ((dj == d) & (j < kio))
        rank = rank + before.astype(jnp.int32)
    mask = rank < _P
    pos = jnp.zeros((_R, _K, _B), jnp.int32)
    for j in range(_K):
        mj = lax.slice_in_dim(mask, j, j + 1, axis=1)
        pos = pos + (mj & (j < kio)).astype(jnp.int32)
    pio = lax.broadcasted_iota(jnp.int32, (_R, _P, _B), 1)
    idsf = ni_ref[...].astype(jnp.float32)
    ch = jnp.zeros((_R, _P, _B), jnp.float32)
    for k in range(_K):
        sel = ((lax.slice_in_dim(pos, k, k + 1, axis=1) == pio) &
               lax.slice_in_dim(mask, k, k + 1, axis=1))
        ch = ch + lax.slice_in_dim(idsf, k, k + 1, axis=1) * sel.astype(jnp.float32)
    out_ref[...] = jnp.transpose(ch, (0, 2, 1)).astype(jnp.int32)


def _select_call(ns3, sself, neigh_t):
    return pl.pallas_call(
        _select_body,
        out_shape=jax.ShapeDtypeStruct((_R, _B, _P), jnp.int32),
    )(ns3, sself, neigh_t)


_NBUF = 2


def _g2_body(chosen_hbm, feat_hbm, agg_out,
             cidx_a, cidx_b, rows_a, rows_b, aggc,
             sem_a, sem_b):
    wid = lax.axis_index("s") * _NC + lax.axis_index("c")
    nrow = _SEG_CHUNK * _P
    bufs = ((cidx_a, rows_a, sem_a), (cidx_b, rows_b, sem_b))

    def start(c):
        cidx, rows, sem = bufs[c % _NBUF]
        base = wid * _SEG_W * _P + c * nrow
        pltpu.sync_copy(chosen_hbm.at[pl.ds(base, nrow)], cidx)
        return pltpu.async_copy(feat_hbm.at[cidx], rows, sem)

    cps = [start(0)]
    for c in range(_CHUNKS):
        cps[0].wait()
        cps.pop(0)
        if c + _NBUF - 1 < _CHUNKS:
            cps.append(start(c + _NBUF - 1))
        rows = bufs[c % _NBUF][1]

        def seg(s, carry2):
            def jstep(j, carry3):
                acc = rows[s * _P, pl.ds(j * _L, _L)]
                for p in range(1, _P):
                    acc = acc + rows[s * _P + p, pl.ds(j * _L, _L)]
                aggc[s, pl.ds(j * _L, _L)] = acc * (1.0 / _P)
                return carry3
            return lax.fori_loop(0, _D // _L, jstep, carry2)
        lax.fori_loop(0, _SEG_CHUNK, seg, 0)
        pltpu.sync_copy(aggc, agg_out.at[pl.ds(wid * _SEG_W + c * _SEG_CHUNK,
                                               _SEG_CHUNK)])


def _g2_call(chosen_flat, features):
    fn = pl.kernel(
        _g2_body,
        out_type=jax.ShapeDtypeStruct((_SEG, _D), jnp.float32),
        mesh=_sc_mesh(),
        scratch_types=[
            pltpu.VMEM((_SEG_CHUNK * _P,), jnp.int32),
            pltpu.VMEM((_SEG_CHUNK * _P,), jnp.int32),
            pltpu.VMEM((_SEG_CHUNK * _P, _D), jnp.float32),
            pltpu.VMEM((_SEG_CHUNK * _P, _D), jnp.float32),
            pltpu.VMEM((_SEG_CHUNK, _D), jnp.float32),
            pltpu.SemaphoreType.DMA,
            pltpu.SemaphoreType.DMA,
        ],
        compiler_params=_SC_PARAMS,
    )
    return fn(chosen_flat, features)


def _final_body(self_ref, agg_ref, wi_ref, wn_ref, wt_ref, out_ref):
    sf = self_ref[...]
    h = jnp.dot(sf, wn_ref[0:_D, :], preferred_element_type=jnp.float32)
    for r in range(_R):
        wr = wi_ref[r]
        ir = jnp.dot(sf, wr[0:_D, :], preferred_element_type=jnp.float32)
        ir = ir + jnp.dot(agg_ref[r], wr[_D:2 * _D, :],
                          preferred_element_type=jnp.float32)
        ir = jnp.maximum(ir, 0.0)
        h = h + jnp.dot(ir, wn_ref[_D + r * _E:_D + (r + 1) * _E, :],
                        preferred_element_type=jnp.float32)
    inter = jnp.maximum(h, 0.0)
    out_ref[...] = lax.dot_general(inter, wt_ref[...],
                                   (((1,), (1,)), ((), ())),
                                   preferred_element_type=jnp.float32)


def _final_call(self_rows, agg3, W_intra, W_inter, weight):
    return pl.pallas_call(
        _final_body,
        out_shape=jax.ShapeDtypeStruct((_B, _C), jnp.float32),
    )(self_rows, agg3, W_intra, W_inter, weight)


def kernel(nodes, labels, neigh_idx, features, train_pos,
           W_label, W_intra, W_inter, weight):
    nodes = nodes.astype(jnp.int32)
    neigh_idx = neigh_idx.astype(jnp.int32)
    all_scores = _all_scores(features, W_label)
    neigh_t = neigh_idx.transpose(0, 2, 1)
    neigh_flat = neigh_t.reshape(-1)
    nsc, ssc, self_rows = _g1_call(all_scores, neigh_flat, nodes, features)
    label_scores = ssc.transpose(0, 2, 1).reshape(_B, 2)
    sself = label_scores[:, 1].reshape(1, _B)
    ns3 = nsc.reshape(_R, _K, _B)
    chosen = _select_call(ns3, sself, neigh_t)
    agg = _g2_call(chosen.reshape(-1), features)
    scores = _final_call(self_rows, agg.reshape(_R, _B, _D),
                         W_intra, W_inter, weight)
    return scores, label_scores


# speedup vs baseline: 1.1105x; 1.0833x over previous
import jax
import jax.numpy as jnp
from jax import lax
from jax.experimental import pallas as pl
from jax.experimental.pallas import tpu as pltpu
from jax.experimental.pallas import tpu_sc as plsc

_N = 10000
_D = 256
_B = 1024
_K = 32
_R = 3
_P = 16
_E = 1024
_C = 2

_NC, _NS, _L = 2, 16, 16
_NW = _NC * _NS

_NEIGH = _R * _B * _K
_NB_W = _NEIGH // _NW
_ND_W = _B // _NW
_SEG = _R * _B
_SEG_W = _SEG // _NW
_SEG_CHUNK = 12
_CHUNKS = _SEG_W // _SEG_CHUNK


def _sc_mesh():
    return plsc.VectorSubcoreMesh(
        core_axis_name="c", subcore_axis_name="s",
        num_cores=_NC, num_subcores=_NS)


_SC_PARAMS = pltpu.CompilerParams(
    needs_layout_passes=False, use_tc_tiling_on_sc=False)


def _scores_body(feat_ref, wl_ref, out_ref):
    out_ref[...] = lax.dot_general(wl_ref[...], feat_ref[...],
                                   (((1,), (1,)), ((), ())),
                                   preferred_element_type=jnp.float32)


def _all_scores(features, w1):
    return pl.pallas_call(
        _scores_body,
        out_shape=jax.ShapeDtypeStruct((1, _N), jnp.float32),
    )(features, w1)


def _fused_body(scores_hbm, neigh_hbm, nodes_hbm, feat_hbm,
                agg_out, srow_out,
                tbl1, ntile, nidx, cidx_a, cidx_b, sidx, srows,
                rows_a, rows_b, aggc, sem_s, sem_a, sem_b):
    wid = lax.axis_index("s") * _NC + lax.axis_index("c")
    pltpu.sync_copy(scores_hbm.at[0], tbl1)
    pltpu.sync_copy(nodes_hbm, ntile)
    pltpu.sync_copy(nodes_hbm.at[pl.ds(wid * _ND_W, _ND_W)], sidx)
    cp_self = pltpu.async_copy(feat_hbm.at[sidx], srows, sem_s)
    pltpu.sync_copy(neigh_hbm.at[pl.ds(wid * _NB_W, _NB_W)], nidx)

    nrow = _SEG_CHUNK * _P
    bufs = ((cidx_a, rows_a, sem_a), (cidx_b, rows_b, sem_b))

    def select_chunk(c):
        cidx = bufs[c % 2][0]

        def seg_sel(i, carry):
            s = c * _SEG_CHUNK + i
            b = lax.rem(wid * _SEG_W + s, _B)
            bvec = jnp.full((_L,), b, jnp.int32)
            nidv = plsc.load_gather(ntile, [bvec])
            sv = plsc.load_gather(tbl1, [nidv])
            ids_a = nidx[pl.ds(s * _K, _L)]
            ids_b = nidx[pl.ds(s * _K + _L, _L)]
            da = jnp.abs(plsc.load_gather(tbl1, [ids_a]) - sv)
            db = jnp.abs(plsc.load_gather(tbl1, [ids_b]) - sv)
            ka, va = plsc.sort_key_val(da, ids_a)
            kb, vb = plsc.sort_key_val(db, ids_b)
            krb = lax.rev(kb, (0,))
            vrb = lax.rev(vb, (0,))
            lo = jnp.where(ka <= krb, va, vrb)
            cidx[pl.ds(i * _P, _P)] = lo
            return carry
        lax.fori_loop(0, _SEG_CHUNK, seg_sel, 0)

    def start(c):
        cidx, rows, sem = bufs[c % 2]
        return pltpu.async_copy(feat_hbm.at[cidx], rows, sem)

    select_chunk(0)
    cps = [start(0)]
    for c in range(_CHUNKS):
        if c + 1 < _CHUNKS:
            select_chunk(c + 1)
            cps.append(start(c + 1))
        cps.pop(0).wait()
        rows = bufs[c % 2][1]

        def seg(s2, carry2):
            def jstep(j, carry3):
                acc = rows[s2 * _P, pl.ds(j * _L, _L)]
                for p in range(1, _P):
                    acc = acc + rows[s2 * _P + p, pl.ds(j * _L, _L)]
                aggc[s2, pl.ds(j * _L, _L)] = acc * (1.0 / _P)
                return carry3
            return lax.fori_loop(0, _D // _L, jstep, carry2)
        lax.fori_loop(0, _SEG_CHUNK, seg, 0)
        pltpu.sync_copy(aggc, agg_out.at[pl.ds(wid * _SEG_W + c * _SEG_CHUNK,
                                               _SEG_CHUNK)])

    cp_self.wait()
    pltpu.sync_copy(srows, srow_out.at[pl.ds(wid * _ND_W, _ND_W)])


def _fused_call(all_scores, neigh_flat, nodes, features):
    fn = pl.kernel(
        _fused_body,
        out_type=(
            jax.ShapeDtypeStruct((_SEG, _D), jnp.float32),
            jax.ShapeDtypeStruct((_B, _D), jnp.float32),
        ),
        mesh=_sc_mesh(),
        scratch_types=[
            pltpu.VMEM((_N,), jnp.float32),
            pltpu.VMEM((_B,), jnp.int32),
            pltpu.VMEM((_NB_W,), jnp.int32),
            pltpu.VMEM((_SEG_CHUNK * _P,), jnp.int32),
            pltpu.VMEM((_SEG_CHUNK * _P,), jnp.int32),
            pltpu.VMEM((_ND_W,), jnp.int32),
            pltpu.VMEM((_ND_W, _D), jnp.float32),
            pltpu.VMEM((_SEG_CHUNK * _P, _D), jnp.float32),
            pltpu.VMEM((_SEG_CHUNK * _P, _D), jnp.float32),
            pltpu.VMEM((_SEG_CHUNK, _D), jnp.float32),
            pltpu.SemaphoreType.DMA,
            pltpu.SemaphoreType.DMA,
            pltpu.SemaphoreType.DMA,
        ],
        compiler_params=_SC_PARAMS,
    )
    return fn(all_scores, neigh_flat, nodes, features)


def _final_body(self_ref, agg_ref, wl_ref, wi_ref, wn_ref, wt_ref,
                out_ref, ls_ref):
    sf = self_ref[...]
    ls_ref[...] = jnp.dot(sf, wl_ref[...],
                          preferred_element_type=jnp.float32)
    h = jnp.dot(sf, wn_ref[0:_D, :], preferred_element_type=jnp.float32)
    for r in range(_R):
        wr = wi_ref[r]
        ir = jnp.dot(sf, wr[0:_D, :], preferred_element_type=jnp.float32)
        ir = ir + jnp.dot(agg_ref[r], wr[_D:2 * _D, :],
                          preferred_element_type=jnp.float32)
        ir = jnp.maximum(ir, 0.0)
        h = h + jnp.dot(ir, wn_ref[_D + r * _E:_D + (r + 1) * _E, :],
                        preferred_element_type=jnp.float32)
    inter = jnp.maximum(h, 0.0)
    out_ref[...] = lax.dot_general(inter, wt_ref[...],
                                   (((1,), (1,)), ((), ())),
                                   preferred_element_type=jnp.float32)


def _final_call(self_rows, agg3, W_label, W_intra, W_inter, weight):
    return pl.pallas_call(
        _final_body,
        out_shape=(
            jax.ShapeDtypeStruct((_B, _C), jnp.float32),
            jax.ShapeDtypeStruct((_B, _C), jnp.float32),
        ),
    )(self_rows, agg3, W_label, W_intra, W_inter, weight)


def kernel(nodes, labels, neigh_idx, features, train_pos,
           W_label, W_intra, W_inter, weight):
    nodes = nodes.astype(jnp.int32)
    neigh_idx = neigh_idx.astype(jnp.int32)
    score1 = _all_scores(features, W_label[:, 1:2].T)
    agg, self_rows = _fused_call(score1, neigh_idx.reshape(-1),
                                 nodes, features)
    scores, label_scores = _final_call(self_rows, agg.reshape(_R, _B, _D),
                                       W_label, W_intra, W_inter, weight)
    return scores, label_scores
